# baseline (speedup 1.0000x reference)
import functools

import jax
import jax.numpy as jnp
from jax import lax
from jax.experimental import pallas as pl
from jax.experimental.pallas import tpu as pltpu
from jax.experimental.pallas import tpu_sc as plsc

N_NODES = 10000
D_IN = 128
D_HID = 64
N_EDGES = 320000

NC = 2
NS = 16
NW = NC * NS
EPW = N_EDGES // NW
CH = 125
NCHUNK = EPW // CH
NB = 8
CHR = 400
NRCH = N_NODES // CHR
DEGW = 16

_mesh = plsc.VectorSubcoreMesh(core_axis_name="c", subcore_axis_name="s")
_sc_params = pltpu.CompilerParams(use_tc_tiling_on_sc=False)


def _sc_degree(dst3, ones_rows, zero_rows):
    @functools.partial(
        pl.kernel,
        mesh=_mesh,
        out_type=(jax.ShapeDtypeStruct((N_NODES, D_HID), jnp.float32),
                  jax.ShapeDtypeStruct((N_NODES, D_HID), jnp.float32)),
        compiler_params=_sc_params,
        scratch_types=[
            pltpu.VMEM((NCHUNK, CH), jnp.int32),
            pltpu.VMEM((CH, DEGW), jnp.float32),
            pltpu.VMEM((CHR, DEGW), jnp.float32),
            pltpu.VMEM((CHR, D_HID), jnp.float32),
            pltpu.VMEM_SHARED((N_NODES, DEGW), jnp.float32),
            pltpu.SemaphoreType.DMA,
        ],
    )
    def k(dst_hbm, ones_hbm, z_hbm, out0_hbm, out1_hbm, dst_v, ones_v, v16,
          v64, acc, sem):
        c = lax.axis_index("c")
        s = lax.axis_index("s")
        wid = c * NS + s
        for k in range(NRCH):
            @pl.when(s == (k % NS))
            def _():
                pltpu.sync_copy(z_hbm, acc.at[pl.ds(k * CHR, CHR)])
        pltpu.sync_copy(ones_hbm, ones_v)
        pltpu.sync_copy(dst_hbm.at[wid], dst_v)
        plsc.subcore_barrier()

        def fire(ci, carry):
            pltpu.async_copy(ones_v, acc.at[dst_v.at[ci]], sem, add=True)
            return carry

        def drain(ci, carry):
            pltpu.make_async_copy(ones_v, acc.at[dst_v.at[ci]], sem).wait()
            return carry

        lax.fori_loop(0, NCHUNK, fire, 0)
        lax.fori_loop(0, NCHUNK, drain, 0)
        plsc.subcore_barrier()
        for k in range(NRCH):
            @pl.when(s == (k % NS))
            def _():
                pltpu.sync_copy(acc.at[pl.ds(k * CHR, CHR)], v16)

                def rowbody(r, carry):
                    v = v16[r]
                    for q in range(4):
                        v64[r, pl.ds(q * DEGW, DEGW)] = v
                    return carry

                lax.fori_loop(0, CHR, rowbody, 0)

            @pl.when((s == (k % NS)) & (c == 0))
            def _():
                pltpu.sync_copy(v64, out0_hbm.at[pl.ds(k * CHR, CHR)])

            @pl.when((s == (k % NS)) & (c == 1))
            def _():
                pltpu.sync_copy(v64, out1_hbm.at[pl.ds(k * CHR, CHR)])

    return k(dst3, ones_rows, zero_rows)


def _sc_seg_sum(y, src3, dst3, zero_rows):
    @functools.partial(
        pl.kernel,
        mesh=_mesh,
        out_type=(jax.ShapeDtypeStruct((N_NODES, D_HID), jnp.float32),
                  jax.ShapeDtypeStruct((N_NODES, D_HID), jnp.float32)),
        compiler_params=_sc_params,
        scratch_types=[
            pltpu.VMEM((NCHUNK, CH), jnp.int32),
            pltpu.VMEM((NCHUNK, CH), jnp.int32),
            pltpu.VMEM((NB, CH, D_HID), jnp.float32),
            pltpu.VMEM_SHARED((N_NODES, D_HID), jnp.float32),
        ] + [pltpu.SemaphoreType.DMA] * (2 * NB),
    )
    def k(y_hbm, src_hbm, dst_hbm, z_hbm, out0_hbm, out1_hbm, src_v, dst_v,
          gbuf, acc, *sems):
        semg, semsc = sems[:NB], sems[NB:]
        c = lax.axis_index("c")
        s = lax.axis_index("s")
        wid = c * NS + s
        for k in range(NRCH):
            @pl.when(s == (k % NS))
            def _():
                pltpu.sync_copy(z_hbm, acc.at[pl.ds(k * CHR, CHR)])
        pltpu.sync_copy(src_hbm.at[wid], src_v)
        pltpu.sync_copy(dst_hbm.at[wid], dst_v)
        plsc.subcore_barrier()

        def start_g(ci, b):
            pltpu.async_copy(y_hbm.at[src_v.at[ci]], gbuf.at[b], semg[b])

        def wait_g(ci, b):
            pltpu.make_async_copy(y_hbm.at[src_v.at[ci]], gbuf.at[b],
                                  semg[b]).wait()

        def start_s(ci, b):
            pltpu.async_copy(gbuf.at[b], acc.at[dst_v.at[ci]], semsc[b],
                             add=True)

        def wait_s(ci, b):
            pltpu.make_async_copy(gbuf.at[b], acc.at[dst_v.at[ci]],
                                  semsc[b]).wait()

        for b in range(NB):
            start_g(b, b)

        def body(j, carry):
            ci = j * NB
            for b in range(NB):
                wait_g(ci + b, b)
                start_s(ci + b, b)
            for b in range(NB):
                wait_s(ci + b, b)
                start_g(ci + NB + b, b)
            return carry

        lax.fori_loop(0, NCHUNK // NB - 1, body, 0)
        ci = NCHUNK - NB
        for b in range(NB):
            wait_g(ci + b, b)
            start_s(ci + b, b)
        for b in range(NB):
            wait_s(ci + b, b)
        plsc.subcore_barrier()
        for k in range(NRCH):
            @pl.when((s == (k % NS)) & (c == 0))
            def _():
                pltpu.sync_copy(acc.at[pl.ds(k * CHR, CHR)],
                                out0_hbm.at[pl.ds(k * CHR, CHR)])

            @pl.when((s == (k % NS)) & (c == 1))
            def _():
                pltpu.sync_copy(acc.at[pl.ds(k * CHR, CHR)],
                                out1_hbm.at[pl.ds(k * CHR, CHR)])

    return k(y, src3, dst3, zero_rows)


_R = 2000
_RP = _R // 2
_NP = N_NODES // 2


def _paired_spec():
    return pl.BlockSpec((_RP, 128), lambda i: (i, 0))


def _tc_first(xp, W1bd, dp0p, dp1p):
    def body(x_ref, w_ref, dp0_ref, dp1_ref, y_ref, dinv_ref):
        dinvp = lax.rsqrt(dp0_ref[...] + dp1_ref[...] + 1.0)
        y_ref[...] = jnp.dot(x_ref[...], w_ref[...],
                             preferred_element_type=jnp.float32) * dinvp
        dinv_ref[...] = dinvp

    return pl.pallas_call(
        body,
        grid=(N_NODES // _R,),
        in_specs=[
            pl.BlockSpec((_RP, 2 * D_IN), lambda i: (i, 0)),
            pl.BlockSpec((2 * D_IN, 128), lambda i: (0, 0)),
            _paired_spec(),
            _paired_spec(),
        ],
        out_specs=(_paired_spec(), _paired_spec()),
        out_shape=(jax.ShapeDtypeStruct((_NP, 128), jnp.float32),
                   jax.ShapeDtypeStruct((_NP, 128), jnp.float32)),
    )(xp, W1bd, dp0p, dp1p)


def _tc_mid(a0p, a1p, yp, dinvp, bp, Wbd):
    def body(a0_ref, a1_ref, y_ref, dinv_ref, b_ref, w_ref, o_ref):
        dinv = dinv_ref[...]
        h = jnp.maximum(
            dinv * (a0_ref[...] + a1_ref[...] + y_ref[...]) + b_ref[...], 0.0)
        o_ref[...] = (
            jnp.dot(h, w_ref[...], preferred_element_type=jnp.float32) * dinv
        )

    return pl.pallas_call(
        body,
        grid=(N_NODES // _R,),
        in_specs=[
            _paired_spec(),
            _paired_spec(),
            _paired_spec(),
            _paired_spec(),
            pl.BlockSpec((1, 128), lambda i: (0, 0)),
            pl.BlockSpec((128, 128), lambda i: (0, 0)),
        ],
        out_specs=_paired_spec(),
        out_shape=jax.ShapeDtypeStruct((_NP, 128), jnp.float32),
    )(a0p, a1p, yp, dinvp, bp, Wbd)


def _tc_head(a0p, a1p, yp, dinvp, bp, Whbd, bhp):
    def body(a0_ref, a1_ref, y_ref, dinv_ref, b_ref, w_ref, bh_ref, o_ref):
        h = (dinv_ref[...] * (a0_ref[...] + a1_ref[...] + y_ref[...])
             + b_ref[...])
        o_ref[...] = (
            jnp.dot(h, w_ref[...], preferred_element_type=jnp.float32)
            + bh_ref[...]
        )

    return pl.pallas_call(
        body,
        grid=(N_NODES // _R,),
        in_specs=[
            _paired_spec(),
            _paired_spec(),
            _paired_spec(),
            _paired_spec(),
            pl.BlockSpec((1, 128), lambda i: (0, 0)),
            pl.BlockSpec((128, 2), lambda i: (0, 0)),
            pl.BlockSpec((1, 2), lambda i: (0, 0)),
        ],
        out_specs=pl.BlockSpec((_RP, 2), lambda i: (i, 0)),
        out_shape=jax.ShapeDtypeStruct((_NP, 2), jnp.float32),
    )(a0p, a1p, yp, dinvp, bp, Whbd, bhp)


def _blockdiag(W):
    k, n = W.shape
    bd = jnp.zeros((2 * k, 2 * n), jnp.float32)
    return bd.at[:k, :n].set(W).at[k:, n:].set(W)


def kernel(x, edge_index, W1, b1, W2, b2, W3, b3, Wh, bh):
    dst3 = edge_index[1].astype(jnp.int32).reshape(NW, NCHUNK, CH)
    src3 = (lax.optimization_barrier(edge_index)[0]
            .astype(jnp.int32).reshape(NW, NCHUNK, CH))
    z_deg = jnp.zeros((CHR, DEGW), jnp.float32)
    z_hid = jnp.zeros((CHR, D_HID), jnp.float32)
    ones_rows = jnp.ones((CH, DEGW), jnp.float32)

    dp0, dp1 = _sc_degree(dst3, ones_rows, z_deg)
    pair = lambda a: a.reshape(_NP, 128)
    unpair = lambda a: a.reshape(N_NODES, D_HID)
    xp = x.reshape(_NP, 2 * D_IN)
    y1p, dinvp = _tc_first(xp, _blockdiag(W1), pair(dp0), pair(dp1))
    b1p = jnp.tile(b1, 2).reshape(1, 128)
    b2p = jnp.tile(b2, 2).reshape(1, 128)
    b3p = jnp.tile(b3, 2).reshape(1, 128)
    A1a, A1b = _sc_seg_sum(unpair(y1p), src3, dst3, z_hid)
    y2p = _tc_mid(pair(A1a), pair(A1b), y1p, dinvp, b1p, _blockdiag(W2))
    A2a, A2b = _sc_seg_sum(unpair(y2p), src3, dst3, z_hid)
    y3p = _tc_mid(pair(A2a), pair(A2b), y2p, dinvp, b2p, _blockdiag(W3))
    A3a, A3b = _sc_seg_sum(unpair(y3p), src3, dst3, z_hid)

    outp = _tc_head(pair(A3a), pair(A3b), y3p, dinvp, b3p, _blockdiag(Wh),
                    jnp.tile(bh, 2).reshape(1, 2))
    return outp.reshape(N_NODES)


# speedup vs baseline: 1.0100x; 1.0100x over previous
import functools

import jax
import jax.numpy as jnp
from jax import lax
from jax.experimental import pallas as pl
from jax.experimental.pallas import tpu as pltpu
from jax.experimental.pallas import tpu_sc as plsc

N_NODES = 10000
D_IN = 128
D_HID = 64
N_EDGES = 320000

NC = 2
NS = 16
NW = NC * NS
EPW = N_EDGES // NW
CH = 125
NCHUNK = EPW // CH
NB = 8
CHR = 400
NRCH = N_NODES // CHR
DEGW = 16

_mesh = plsc.VectorSubcoreMesh(core_axis_name="c", subcore_axis_name="s")
_sc_params = pltpu.CompilerParams(use_tc_tiling_on_sc=False)


def _sc_degree(dst3, ones_rows, zero_rows):
    @functools.partial(
        pl.kernel,
        mesh=_mesh,
        out_type=(jax.ShapeDtypeStruct((N_NODES, D_HID), jnp.float32),
                  jax.ShapeDtypeStruct((N_NODES, D_HID), jnp.float32)),
        compiler_params=_sc_params,
        scratch_types=[
            pltpu.VMEM((NCHUNK, CH), jnp.int32),
            pltpu.VMEM((CH, DEGW), jnp.float32),
            pltpu.VMEM((CHR, DEGW), jnp.float32),
            pltpu.VMEM((CHR, D_HID), jnp.float32),
            pltpu.VMEM_SHARED((N_NODES, DEGW), jnp.float32),
            pltpu.SemaphoreType.DMA,
        ],
    )
    def k(dst_hbm, ones_hbm, z_hbm, out0_hbm, out1_hbm, dst_v, ones_v, v16,
          v64, acc, sem):
        c = lax.axis_index("c")
        s = lax.axis_index("s")
        wid = c * NS + s
        for k in range(NRCH):
            @pl.when(s == (k % NS))
            def _():
                pltpu.sync_copy(z_hbm, acc.at[pl.ds(k * CHR, CHR)])
        pltpu.sync_copy(ones_hbm, ones_v)
        pltpu.sync_copy(dst_hbm.at[wid], dst_v)
        plsc.subcore_barrier()

        def fire(ci, carry):
            pltpu.async_copy(ones_v, acc.at[dst_v.at[ci]], sem, add=True)
            return carry

        def drain(ci, carry):
            pltpu.make_async_copy(ones_v, acc.at[dst_v.at[ci]], sem).wait()
            return carry

        lax.fori_loop(0, NCHUNK, fire, 0)
        lax.fori_loop(0, NCHUNK, drain, 0)
        plsc.subcore_barrier()
        for k in range(NRCH):
            @pl.when(s == (k % NS))
            def _():
                pltpu.sync_copy(acc.at[pl.ds(k * CHR, CHR)], v16)

                def rowbody(r, carry):
                    v = v16[r]
                    for q in range(4):
                        v64[r, pl.ds(q * DEGW, DEGW)] = v
                    return carry

                lax.fori_loop(0, CHR, rowbody, 0)

            @pl.when((s == (k % NS)) & (c == 0))
            def _():
                pltpu.sync_copy(v64, out0_hbm.at[pl.ds(k * CHR, CHR)])

            @pl.when((s == (k % NS)) & (c == 1))
            def _():
                pltpu.sync_copy(v64, out1_hbm.at[pl.ds(k * CHR, CHR)])

    return k(dst3, ones_rows, zero_rows)


def _sc_seg_sum(y, src3, dst3, zero_rows):
    @functools.partial(
        pl.kernel,
        mesh=_mesh,
        out_type=(jax.ShapeDtypeStruct((N_NODES, D_HID), jnp.float32),
                  jax.ShapeDtypeStruct((N_NODES, D_HID), jnp.float32)),
        compiler_params=_sc_params,
        scratch_types=[
            pltpu.VMEM((NCHUNK, CH), jnp.int32),
            pltpu.VMEM((NCHUNK, CH), jnp.int32),
            pltpu.VMEM((NB, CH, D_HID), jnp.float32),
            pltpu.VMEM_SHARED((N_NODES, D_HID), jnp.float32),
        ] + [pltpu.SemaphoreType.DMA] * (2 * NB),
    )
    def k(y_hbm, src_hbm, dst_hbm, z_hbm, out0_hbm, out1_hbm, src_v, dst_v,
          gbuf, acc, *sems):
        semg, semsc = sems[:NB], sems[NB:]
        c = lax.axis_index("c")
        s = lax.axis_index("s")
        wid = c * NS + s
        for k in range(NRCH):
            @pl.when(s == (k % NS))
            def _():
                pltpu.sync_copy(z_hbm, acc.at[pl.ds(k * CHR, CHR)])
        pltpu.sync_copy(src_hbm.at[wid], src_v)
        pltpu.sync_copy(dst_hbm.at[wid], dst_v)
        plsc.subcore_barrier()

        def start_g(ci, b):
            pltpu.async_copy(y_hbm.at[src_v.at[ci]], gbuf.at[b], semg[b])

        def wait_g(ci, b):
            pltpu.make_async_copy(y_hbm.at[src_v.at[ci]], gbuf.at[b],
                                  semg[b]).wait()

        def start_s(ci, b):
            pltpu.async_copy(gbuf.at[b], acc.at[dst_v.at[ci]], semsc[b],
                             add=True)

        def wait_s(ci, b):
            pltpu.make_async_copy(gbuf.at[b], acc.at[dst_v.at[ci]],
                                  semsc[b]).wait()

        for b in range(NB):
            start_g(b, b)

        def body(j, carry):
            ci = j * NB
            for b in range(NB):
                wait_g(ci + b, b)
                start_s(ci + b, b)
            for b in range(NB):
                wait_s(ci + b, b)
                start_g(ci + NB + b, b)
            return carry

        lax.fori_loop(0, NCHUNK // NB - 1, body, 0)
        ci = NCHUNK - NB
        for b in range(NB):
            wait_g(ci + b, b)
            start_s(ci + b, b)
        for b in range(NB):
            wait_s(ci + b, b)
        plsc.subcore_barrier()
        for k in range(NRCH):
            @pl.when((s == (k % NS)) & (c == 0))
            def _():
                pltpu.sync_copy(acc.at[pl.ds(k * CHR, CHR)],
                                out0_hbm.at[pl.ds(k * CHR, CHR)])

            @pl.when((s == (k % NS)) & (c == 1))
            def _():
                pltpu.sync_copy(acc.at[pl.ds(k * CHR, CHR)],
                                out1_hbm.at[pl.ds(k * CHR, CHR)])

    return k(y, src3, dst3, zero_rows)


_R = 2000
_RP = _R // 2
_NP = N_NODES // 2


def _paired_spec():
    return pl.BlockSpec((_RP, 128), lambda i: (i, 0))


def _tc_first(xp, W1bd, dp0p, dp1p):
    def body(x_ref, w_ref, dp0_ref, dp1_ref, y_ref, dinv_ref):
        dinvp = lax.rsqrt(dp0_ref[...] + dp1_ref[...] + 1.0)
        y_ref[...] = jnp.dot(x_ref[...], w_ref[...],
                             preferred_element_type=jnp.float32) * dinvp
        dinv_ref[...] = dinvp

    return pl.pallas_call(
        body,
        grid=(N_NODES // _R,),
        in_specs=[
            pl.BlockSpec((_RP, 2 * D_IN), lambda i: (i, 0)),
            pl.BlockSpec((2 * D_IN, 128), lambda i: (0, 0)),
            _paired_spec(),
            _paired_spec(),
        ],
        out_specs=(_paired_spec(), _paired_spec()),
        out_shape=(jax.ShapeDtypeStruct((_NP, 128), jnp.float32),
                   jax.ShapeDtypeStruct((_NP, 128), jnp.float32)),
    )(xp, W1bd, dp0p, dp1p)


def _tc_mid(a0p, a1p, yp, dinvp, bp, Wbd):
    def body(a0_ref, a1_ref, y_ref, dinv_ref, b_ref, w_ref, o_ref):
        dinv = dinv_ref[...]
        h = jnp.maximum(
            dinv * (a0_ref[...] + a1_ref[...] + y_ref[...]) + b_ref[...], 0.0)
        o_ref[...] = (
            jnp.dot(h, w_ref[...], preferred_element_type=jnp.float32) * dinv
        )

    return pl.pallas_call(
        body,
        grid=(N_NODES // _R,),
        in_specs=[
            _paired_spec(),
            _paired_spec(),
            _paired_spec(),
            _paired_spec(),
            pl.BlockSpec((1, 128), lambda i: (0, 0)),
            pl.BlockSpec((128, 128), lambda i: (0, 0)),
        ],
        out_specs=_paired_spec(),
        out_shape=jax.ShapeDtypeStruct((_NP, 128), jnp.float32),
    )(a0p, a1p, yp, dinvp, bp, Wbd)


def _tc_head(a0p, a1p, yp, dinvp, bp, Whbd, bhp):
    def body(a0_ref, a1_ref, y_ref, dinv_ref, b_ref, w_ref, bh_ref, o_ref):
        h = (dinv_ref[...] * (a0_ref[...] + a1_ref[...] + y_ref[...])
             + b_ref[...])
        o_ref[...] = (
            jnp.dot(h, w_ref[...], preferred_element_type=jnp.float32)
            + bh_ref[...]
        )

    return pl.pallas_call(
        body,
        grid=(N_NODES // _R,),
        in_specs=[
            _paired_spec(),
            _paired_spec(),
            _paired_spec(),
            _paired_spec(),
            pl.BlockSpec((1, 128), lambda i: (0, 0)),
            pl.BlockSpec((128, 2), lambda i: (0, 0)),
            pl.BlockSpec((1, 2), lambda i: (0, 0)),
        ],
        out_specs=pl.BlockSpec((_RP, 2), lambda i: (i, 0)),
        out_shape=jax.ShapeDtypeStruct((_NP, 2), jnp.float32),
    )(a0p, a1p, yp, dinvp, bp, Whbd, bhp)


def _blockdiag(W):
    k, n = W.shape
    bd = jnp.zeros((2 * k, 2 * n), jnp.float32)
    return bd.at[:k, :n].set(W).at[k:, n:].set(W)


def kernel(x, edge_index, W1, b1, W2, b2, W3, b3, Wh, bh):
    src3 = edge_index[0].astype(jnp.int32).reshape(NW, NCHUNK, CH)
    dst3 = edge_index[1].astype(jnp.int32).reshape(NW, NCHUNK, CH)
    z_deg = jnp.zeros((CHR, DEGW), jnp.float32)
    z_hid = jnp.zeros((CHR, D_HID), jnp.float32)
    ones_rows = jnp.ones((CH, DEGW), jnp.float32)

    dp0, dp1 = _sc_degree(dst3, ones_rows, z_deg)
    pair = lambda a: a.reshape(_NP, 128)
    unpair = lambda a: a.reshape(N_NODES, D_HID)
    xp = x.reshape(_NP, 2 * D_IN)
    y1p, dinvp = _tc_first(xp, _blockdiag(W1), pair(dp0), pair(dp1))
    b1p = jnp.tile(b1, 2).reshape(1, 128)
    b2p = jnp.tile(b2, 2).reshape(1, 128)
    b3p = jnp.tile(b3, 2).reshape(1, 128)
    A1a, A1b = _sc_seg_sum(unpair(y1p), src3, dst3, z_hid)
    y2p = _tc_mid(pair(A1a), pair(A1b), y1p, dinvp, b1p, _blockdiag(W2))
    A2a, A2b = _sc_seg_sum(unpair(y2p), src3, dst3, z_hid)
    y3p = _tc_mid(pair(A2a), pair(A2b), y2p, dinvp, b2p, _blockdiag(W3))
    A3a, A3b = _sc_seg_sum(unpair(y3p), src3, dst3, z_hid)

    outp = _tc_head(pair(A3a), pair(A3b), y3p, dinvp, b3p, _blockdiag(Wh),
                    jnp.tile(bh, 2).reshape(1, 2))
    return outp.reshape(N_NODES)
